# Initial kernel scaffold; baseline (speedup 1.0000x reference)
import jax
import jax.numpy as jnp
from jax.experimental import pallas as pl


def kernel(length, rel_embeddings):
    raise NotImplementedError("write your pallas kernel here")


# speedup vs baseline: 9.8230x; 9.8230x over previous
import functools

import jax
import jax.numpy as jnp
from jax import lax
from jax.experimental import pallas as pl
from jax.experimental.pallas import tpu as pltpu
from jax.experimental.pallas import tpu_sc as plsc


def _build_sc_kernel(n_rows: int, depth: int):
    L = (n_rows + 1) // 2
    info = plsc.get_sparse_core_info()
    nc, ns = info.num_cores, info.num_subcores
    nw = nc * ns
    rows_per_w = L // nw
    window = 8

    mesh = plsc.VectorSubcoreMesh(core_axis_name="c", subcore_axis_name="s")

    @functools.partial(
        pl.kernel,
        mesh=mesh,
        out_type=jax.ShapeDtypeStruct((L, L, depth), jnp.float32),
        scratch_types=[
            pltpu.VMEM((n_rows, depth), jnp.float32),
            pltpu.SemaphoreType.DMA,
        ],
        compiler_params=pltpu.CompilerParams(use_tc_tiling_on_sc=False),
    )
    def k(table_hbm, out_hbm, tab_v, sem):
        wid = lax.axis_index("s") * nc + lax.axis_index("c")
        base = wid * rows_per_w

        pltpu.sync_copy(table_hbm, tab_v)

        def flip_body(r, carry):
            a = tab_v[r, :]
            b = tab_v[n_rows - 1 - r, :]
            tab_v[r, :] = b
            tab_v[n_rows - 1 - r, :] = a
            return carry

        lax.fori_loop(0, n_rows // 2, flip_body, 0, unroll=4)

        def fire(i):
            row = base + i
            pltpu.async_copy(
                tab_v.at[pl.ds(L - 1 - row, L), :], out_hbm.at[row], sem
            )

        def drain_one():
            pltpu.make_async_copy(
                tab_v.at[pl.ds(0, L), :], out_hbm.at[base], sem
            ).wait()

        for i in range(window):
            fire(i)

        def body(i, carry):
            fire(i)
            drain_one()
            return carry

        lax.fori_loop(window, rows_per_w, body, 0)
        for _ in range(window):
            drain_one()

    return k


def kernel(length, rel_embeddings):
    n_rows, depth = rel_embeddings.shape
    k = _build_sc_kernel(n_rows, depth)
    return k(rel_embeddings)


# speedup vs baseline: 112.3865x; 11.4412x over previous
import functools

import jax
import jax.numpy as jnp
from jax import lax
from jax.experimental import pallas as pl
from jax.experimental.pallas import tpu as pltpu
from jax.experimental.pallas import tpu_sc as plsc


def _build_sc_kernel(n_rows: int, depth: int):
    L = (n_rows + 1) // 2
    n_db = depth // 8
    n_jb = L // 128
    n_q = (L - 128) // 128 + 16
    w_trs = n_rows - 7
    info = plsc.get_sparse_core_info()
    nc, ns = info.num_cores, info.num_subcores
    nw = nc * ns
    res_per_w = 128 // nw
    rows_per_res = L // 128

    mesh = plsc.VectorSubcoreMesh(core_axis_name="c", subcore_axis_name="s")

    @functools.partial(
        pl.kernel,
        mesh=mesh,
        out_type=jax.ShapeDtypeStruct((L, n_db, n_jb, 8, 128), jnp.float32),
        scratch_types=[
            pltpu.VMEM((2, n_q, depth, 128), jnp.float32),
            pltpu.SemaphoreType.DMA,
            pltpu.SemaphoreType.DMA,
            pltpu.SemaphoreType.DMA,
        ],
        compiler_params=pltpu.CompilerParams(use_tc_tiling_on_sc=False),
    )
    def k(trs_hbm, out_hbm, tt_v, semb0, semb1, semo):
        wid = lax.axis_index("s") * nc + lax.axis_index("c")
        semb = (semb0, semb1)

        def fire_build(rho, buf):
            delta = rho % 8
            rho8 = pl.multiple_of(rho - delta, 8)

            def body(q, carry):
                pltpu.async_copy(
                    trs_hbm.at[delta, :, pl.ds(rho8 + 128 * q, 128)],
                    tt_v.at[buf, q],
                    semb[buf],
                )
                return carry

            lax.fori_loop(0, n_q, body, 0)

        def wait_build(buf):
            def body(q, carry):
                pltpu.make_async_copy(
                    trs_hbm.at[0, :, pl.ds(0, 128)], tt_v.at[buf, 0], semb[buf]
                ).wait()
                return carry

            lax.fori_loop(0, n_q, body, 0)

        def fire_outs(rho, buf):
            def body(q0, carry):
                i = (L - 1) - rho - 128 * q0
                for db in range(n_db):
                    pltpu.async_copy(
                        tt_v.at[buf, pl.ds(q0, n_jb), pl.ds(db * 8, 8), :],
                        out_hbm.at[i, db],
                        semo,
                    )
                return carry

            lax.fori_loop(0, rows_per_res, body, 0)

        def drain_outs():
            def body(q0, carry):
                pltpu.make_async_copy(
                    tt_v.at[0, pl.ds(0, n_jb), pl.ds(0, 8), :],
                    out_hbm.at[0, 0],
                    semo,
                ).wait()
                return carry

            lax.fori_loop(0, rows_per_res * n_db, body, 0)

        fire_build(res_per_w * wid, 0)
        for kres in range(res_per_w):
            buf = kres & 1
            rho = res_per_w * wid + kres
            wait_build(buf)
            if kres + 1 < res_per_w:
                fire_build(rho + 1, 1 - buf)
            fire_outs(rho, buf)
            drain_outs()

    return k


def kernel(length, rel_embeddings):
    n_rows, depth = rel_embeddings.shape
    L = (n_rows + 1) // 2
    w_trs = n_rows - 7
    tr = jnp.flip(rel_embeddings, axis=0).T
    trs = jnp.stack([tr[:, s : s + w_trs] for s in range(8)])
    out5 = _build_sc_kernel(n_rows, depth)(trs)
    return jnp.transpose(out5, (0, 2, 4, 1, 3)).reshape(L, L, depth)
